# Initial kernel scaffold; baseline (speedup 1.0000x reference)
import jax
import jax.numpy as jnp
from jax.experimental import pallas as pl


def kernel(img_feat, depth_logits, rots, trans, intrins):
    raise NotImplementedError("write your pallas kernel here")


# speedup vs baseline: 7.6063x; 7.6063x over previous
import functools
import math

import numpy as np
import jax
import jax.numpy as jnp
from jax import lax
from jax.experimental import pallas as pl
from jax.experimental.pallas import tpu as pltpu
from jax.experimental.pallas import tpu_sc as plsc

N, C, D, H, W = 6, 64, 40, 32, 88
HW = H * W
NP = D * HW
BEV = 125
NCELL = BEV * BEV
K = 128
NSUB = 16
HWB = 128
DB = 8


def _build_consts():
    z = np.arange(1.0, 41.0, 1.0, dtype=np.float32) + np.float32(1.0)
    x = np.linspace(0.0, W - 1, W).astype(np.float32)
    y = np.linspace(0.0, H - 1, H).astype(np.float32)
    xz = x[None, :] * z[:, None]
    yz = y[None, :] * z[:, None]
    m1 = np.float32(max(xz.max(), yz.max(), z.max()))
    col0 = xz / m1 * np.float32(100.0) + np.float32(-50.0)
    m2 = np.float32(max(col0.max(), yz.max(), z.max()))
    col1 = yz / m2 * np.float32(100.0) + np.float32(-50.0)
    xi = ((col0 - np.float32(-50.0)) / np.float32(0.8)).astype(np.int32)
    yi = ((col1 - np.float32(-50.0)) / np.float32(0.8)).astype(np.int32)
    zi = ((z - np.float32(-10.0)) / np.float32(20.0)).astype(np.int32)
    valid = ((xi[:, None, :] >= 0) & (xi[:, None, :] < BEV)
             & (yi[:, :, None] >= 0) & (yi[:, :, None] < BEV)
             & (zi[:, None, None] >= 0))
    cell = np.broadcast_to(yi[:, :, None] * BEV + xi[:, None, :],
                           (D, H, W)).reshape(-1)
    vmask = valid.reshape(-1)
    pts = np.nonzero(vmask)[0]
    order = np.argsort(cell[pts], kind="stable")
    perm = pts[order].astype(np.int32)
    scell = cell[pts][order].astype(np.int32)
    nv = perm.size

    split_cell = int(scell[nv // 2])
    i0 = int(np.searchsorted(scell, split_cell, side="left"))
    nr0, nr1 = split_cell, NCELL - split_cell
    nra = max(nr0, nr1) + 1
    dummy = nra - 1

    lists = []
    for base, cnt, cell_base in ((0, i0, 0), (i0, nv - i0, split_cell)):
        q, r = divmod(cnt, NSUB)
        off = base
        for s in range(NSUB):
            take = q + (1 if s < r else 0)
            lists.append((perm[off:off + take],
                          scell[off:off + take] - cell_base))
            off += take
    nchunk = max(1, math.ceil(max(len(g) for g, _ in lists) / K))
    gidx = np.zeros((2 * NSUB, nchunk * K), np.int32)
    sidx = np.full((2 * NSUB, nchunk * K), dummy, np.int32)
    for w_, (g, sc_) in enumerate(lists):
        gidx[w_, :g.size] = g
        sidx[w_, :sc_.size] = sc_
    gidx = gidx.reshape(2 * NSUB * nchunk, K)
    sidx = sidx.reshape(2 * NSUB * nchunk, K)
    return gidx, sidx, nchunk, nr0, nr1, nra, split_cell


_GIDX, _SIDX, NCHUNK, NR0, NR1, NRA, SPLIT = _build_consts()
RPZ = math.ceil(NRA / NSUB)
RPS0 = math.ceil(NR0 / NSUB)
RPS1 = math.ceil(NR1 / NSUB)


def _tc_body(dl_ref, im_ref, f_ref):
    j = pl.program_id(1)
    lo = dl_ref[...]
    m = jnp.max(lo, axis=1, keepdims=True)
    e = jnp.exp(lo - m)
    r = 1.0 / jnp.sum(e, axis=1, keepdims=True)
    lo_j = dl_ref[:, pl.ds(j * DB, DB), :]
    p = jnp.exp(lo_j - m) * r
    im = im_ref[...]
    acc = p[0][:, :, None] * im[0][None, :, :]
    for n in range(1, N):
        acc = acc + p[n][:, :, None] * im[n][None, :, :]
    f_ref[...] = acc


def _tc_features(dl3, imt3):
    return pl.pallas_call(
        _tc_body,
        grid=(HW // HWB, D // DB),
        in_specs=[
            pl.BlockSpec((N, D, HWB), lambda i, j: (0, 0, i)),
            pl.BlockSpec((N, HWB, C), lambda i, j: (0, i, 0)),
        ],
        out_specs=pl.BlockSpec((DB, HWB, C), lambda i, j: (j, i, 0)),
        out_shape=jax.ShapeDtypeStruct((D, HW, C), jnp.float32),
    )(dl3, imt3)


@functools.cache
def _make_sc_scatter():
    @functools.partial(
        pl.kernel,
        mesh=plsc.VectorSubcoreMesh(core_axis_name="c", subcore_axis_name="s"),
        out_type=jax.ShapeDtypeStruct((NCELL, C), jnp.float32),
        scratch_types=[
            pltpu.VMEM((K,), jnp.int32),
            pltpu.VMEM((K,), jnp.int32),
            pltpu.VMEM((K, C), jnp.float32),
            pltpu.VMEM_SHARED((NRA, C), jnp.float32),
            pltpu.SemaphoreType.DMA,
        ],
        compiler_params=pltpu.CompilerParams(use_tc_tiling_on_sc=False),
    )
    def _sc_scatter(f_hbm, gidx_hbm, sidx_hbm, zrows_hbm, out_hbm,
                    gidx_v, sidx_v, rows_v, acc, sem):
        c = lax.axis_index("c")
        s = lax.axis_index("s")
        wid = c * NSUB + s

        zst = jnp.minimum(s * RPZ, NRA - RPZ)
        pltpu.sync_copy(zrows_hbm, acc.at[pl.ds(zst, RPZ)])
        plsc.subcore_barrier()

        def chunk(j, carry):
            row = wid * NCHUNK + j
            pltpu.sync_copy(gidx_hbm.at[row], gidx_v)
            pltpu.sync_copy(sidx_hbm.at[row], sidx_v)
            pltpu.async_copy(f_hbm.at[gidx_v], rows_v, sem).wait()
            pltpu.sync_copy(rows_v, acc.at[sidx_v], add=True)
            return carry

        lax.fori_loop(0, NCHUNK, chunk, 0)
        plsc.subcore_barrier()

        @pl.when(c == 0)
        def _():
            st = jnp.minimum(s * RPS0, NR0 - RPS0)
            pltpu.sync_copy(acc.at[pl.ds(st, RPS0)],
                            out_hbm.at[pl.ds(st, RPS0)])

        @pl.when(c == 1)
        def _():
            st = jnp.minimum(s * RPS1, NR1 - RPS1)
            pltpu.sync_copy(acc.at[pl.ds(st, RPS1)],
                            out_hbm.at[pl.ds(SPLIT + st, RPS1)])

    return _sc_scatter


def kernel(img_feat, depth_logits, rots, trans, intrins):
    del rots, trans, intrins
    imf = img_feat.reshape(N, C, H, W)
    dl3 = depth_logits.reshape(N, D, HW)
    imt3 = jnp.transpose(imf, (0, 2, 3, 1)).reshape(N, HW, C)
    feats = _tc_features(dl3, imt3).reshape(NP, C)
    bev = _make_sc_scatter()(feats, jnp.asarray(_GIDX), jnp.asarray(_SIDX),
                             jnp.zeros((RPZ, C), jnp.float32))
    return jnp.transpose(bev.reshape(BEV, BEV, C), (2, 0, 1))[None]


# speedup vs baseline: 8.9288x; 1.1739x over previous
import functools
import math

import numpy as np
import jax
import jax.numpy as jnp
from jax import lax
from jax.experimental import pallas as pl
from jax.experimental.pallas import tpu as pltpu
from jax.experimental.pallas import tpu_sc as plsc

N, C, D, H, W = 6, 64, 40, 32, 88
HW = H * W
NP = D * HW
BEV = 125
NCELL = BEV * BEV
K = 128
NBUF = 4
NSUB = 16
HWB = 128
DB = 8


def _build_consts():
    z = np.arange(1.0, 41.0, 1.0, dtype=np.float32) + np.float32(1.0)
    x = np.linspace(0.0, W - 1, W).astype(np.float32)
    y = np.linspace(0.0, H - 1, H).astype(np.float32)
    xz = x[None, :] * z[:, None]
    yz = y[None, :] * z[:, None]
    m1 = np.float32(max(xz.max(), yz.max(), z.max()))
    col0 = xz / m1 * np.float32(100.0) + np.float32(-50.0)
    m2 = np.float32(max(col0.max(), yz.max(), z.max()))
    col1 = yz / m2 * np.float32(100.0) + np.float32(-50.0)
    xi = ((col0 - np.float32(-50.0)) / np.float32(0.8)).astype(np.int32)
    yi = ((col1 - np.float32(-50.0)) / np.float32(0.8)).astype(np.int32)
    zi = ((z - np.float32(-10.0)) / np.float32(20.0)).astype(np.int32)
    valid = ((xi[:, None, :] >= 0) & (xi[:, None, :] < BEV)
             & (yi[:, :, None] >= 0) & (yi[:, :, None] < BEV)
             & (zi[:, None, None] >= 0))
    cell = np.broadcast_to(yi[:, :, None] * BEV + xi[:, None, :],
                           (D, H, W)).reshape(-1)
    vmask = valid.reshape(-1)
    pts = np.nonzero(vmask)[0]
    order = np.argsort(cell[pts], kind="stable")
    perm = pts[order].astype(np.int32)
    scell = cell[pts][order].astype(np.int32)
    nv = perm.size

    split_cell = int(scell[nv // 2])
    i0 = int(np.searchsorted(scell, split_cell, side="left"))
    nr0, nr1 = split_cell, NCELL - split_cell
    nra = max(nr0, nr1) + 1
    dummy = nra - 1

    lists = []
    for base, cnt, cell_base in ((0, i0, 0), (i0, nv - i0, split_cell)):
        q, r = divmod(cnt, NSUB)
        off = base
        for s in range(NSUB):
            take = q + (1 if s < r else 0)
            lists.append((perm[off:off + take],
                          scell[off:off + take] - cell_base))
            off += take
    nchunk = max(1, math.ceil(max(len(g) for g, _ in lists) / K))
    nchunk = math.ceil(nchunk / NBUF) * NBUF
    gidx = np.zeros((2 * NSUB, nchunk * K), np.int32)
    sidx = np.full((2 * NSUB, nchunk * K), dummy, np.int32)
    for w_, (g, sc_) in enumerate(lists):
        gidx[w_, :g.size] = g
        sidx[w_, :sc_.size] = sc_
    gidx = gidx.reshape(2 * NSUB * nchunk, K)
    sidx = sidx.reshape(2 * NSUB * nchunk, K)
    return gidx, sidx, nchunk, nr0, nr1, nra, split_cell


_GIDX, _SIDX, NCHUNK, NR0, NR1, NRA, SPLIT = _build_consts()
RPZ = math.ceil(NRA / NSUB)
RPS0 = math.ceil(NR0 / NSUB)
RPS1 = math.ceil(NR1 / NSUB)


def _tc_body(dl_ref, im_ref, f_ref):
    j = pl.program_id(1)
    lo = dl_ref[...]
    m = jnp.max(lo, axis=1, keepdims=True)
    e = jnp.exp(lo - m)
    r = 1.0 / jnp.sum(e, axis=1, keepdims=True)
    lo_j = dl_ref[:, pl.ds(j * DB, DB), :]
    p = jnp.exp(lo_j - m) * r
    im = im_ref[...]
    acc = p[0][:, :, None] * im[0][None, :, :]
    for n in range(1, N):
        acc = acc + p[n][:, :, None] * im[n][None, :, :]
    f_ref[...] = acc.reshape(DB, HWB * C)


def _tc_features(dl3, imt3):
    return pl.pallas_call(
        _tc_body,
        grid=(HW // HWB, D // DB),
        in_specs=[
            pl.BlockSpec((N, D, HWB), lambda i, j: (0, 0, i)),
            pl.BlockSpec((N, HWB, C), lambda i, j: (0, i, 0)),
        ],
        out_specs=pl.BlockSpec((DB, HWB * C), lambda i, j: (j, i)),
        out_shape=jax.ShapeDtypeStruct((D, HW * C), jnp.float32),
    )(dl3, imt3)


@functools.cache
def _make_sc_scatter():
    @functools.partial(
        pl.kernel,
        mesh=plsc.VectorSubcoreMesh(core_axis_name="c", subcore_axis_name="s"),
        out_type=jax.ShapeDtypeStruct((NCELL, C), jnp.float32),
        scratch_types=[
            pltpu.VMEM((NCHUNK, K), jnp.int32),
            pltpu.VMEM((NCHUNK, K), jnp.int32),
            pltpu.VMEM((NBUF, K, C), jnp.float32),
            pltpu.VMEM_SHARED((NRA, C), jnp.float32),
            pltpu.SemaphoreType.DMA,
            pltpu.SemaphoreType.DMA,
            pltpu.SemaphoreType.DMA,
            pltpu.SemaphoreType.DMA,
        ],
        compiler_params=pltpu.CompilerParams(use_tc_tiling_on_sc=False),
    )
    def _sc_scatter(f_hbm, gidx_hbm, sidx_hbm, zrows_hbm, out_hbm,
                    gidx_all, sidx_all, rows, acc, s0, s1, s2, s3):
        sems = [s0, s1, s2, s3]
        c = lax.axis_index("c")
        s = lax.axis_index("s")
        wid = c * NSUB + s

        pltpu.sync_copy(gidx_hbm.at[pl.ds(wid * NCHUNK, NCHUNK)], gidx_all)
        pltpu.sync_copy(sidx_hbm.at[pl.ds(wid * NCHUNK, NCHUNK)], sidx_all)

        def gather(j, b):
            return pltpu.make_async_copy(f_hbm.at[gidx_all.at[j]],
                                         rows.at[b], sems[b])

        for b in range(NBUF):
            gather(b, b).start()

        zst = jnp.minimum(s * RPZ, NRA - RPZ)
        pltpu.sync_copy(zrows_hbm, acc.at[pl.ds(zst, RPZ)])
        plsc.subcore_barrier()

        @pl.loop(0, NCHUNK, step=NBUF)
        def _(j0):
            for b in range(NBUF):
                j = j0 + b
                gather(j, b).wait()
                pltpu.sync_copy(rows.at[b], acc.at[sidx_all.at[j]], add=True)
                nj = j + NBUF

                @pl.when(nj < NCHUNK)
                def _():
                    gather(nj, b).start()

        plsc.subcore_barrier()

        @pl.when(c == 0)
        def _():
            st = jnp.minimum(s * RPS0, NR0 - RPS0)
            pltpu.sync_copy(acc.at[pl.ds(st, RPS0)],
                            out_hbm.at[pl.ds(st, RPS0)])

        @pl.when(c == 1)
        def _():
            st = jnp.minimum(s * RPS1, NR1 - RPS1)
            pltpu.sync_copy(acc.at[pl.ds(st, RPS1)],
                            out_hbm.at[pl.ds(SPLIT + st, RPS1)])

    return _sc_scatter


def kernel(img_feat, depth_logits, rots, trans, intrins):
    del rots, trans, intrins
    imf = img_feat.reshape(N, C, H, W)
    dl3 = depth_logits.reshape(N, D, HW)
    imt3 = jnp.transpose(imf, (0, 2, 3, 1)).reshape(N, HW, C)
    feats = jnp.reshape(_tc_features(dl3, imt3), (NP, C))
    bev = _make_sc_scatter()(feats, jnp.asarray(_GIDX), jnp.asarray(_SIDX),
                             jnp.zeros((RPZ, C), jnp.float32))
    return jnp.transpose(bev.reshape(BEV, BEV, C), (2, 0, 1))[None]


# speedup vs baseline: 9.0824x; 1.0172x over previous
import functools
import math

import numpy as np
import jax
import jax.numpy as jnp
from jax import lax
from jax.experimental import pallas as pl
from jax.experimental.pallas import tpu as pltpu
from jax.experimental.pallas import tpu_sc as plsc

N, C, D, H, W = 6, 64, 40, 32, 88
HW = H * W
NP = D * HW
BEV = 125
NCELL = BEV * BEV
K = 128
NBUF = 4
NSUB = 16
HWB = 128
DB = 8


def _build_consts():
    z = np.arange(1.0, 41.0, 1.0, dtype=np.float32) + np.float32(1.0)
    x = np.linspace(0.0, W - 1, W).astype(np.float32)
    y = np.linspace(0.0, H - 1, H).astype(np.float32)
    xz = x[None, :] * z[:, None]
    yz = y[None, :] * z[:, None]
    m1 = np.float32(max(xz.max(), yz.max(), z.max()))
    col0 = xz / m1 * np.float32(100.0) + np.float32(-50.0)
    m2 = np.float32(max(col0.max(), yz.max(), z.max()))
    col1 = yz / m2 * np.float32(100.0) + np.float32(-50.0)
    xi = ((col0 - np.float32(-50.0)) / np.float32(0.8)).astype(np.int32)
    yi = ((col1 - np.float32(-50.0)) / np.float32(0.8)).astype(np.int32)
    zi = ((z - np.float32(-10.0)) / np.float32(20.0)).astype(np.int32)
    valid = ((xi[:, None, :] >= 0) & (xi[:, None, :] < BEV)
             & (yi[:, :, None] >= 0) & (yi[:, :, None] < BEV)
             & (zi[:, None, None] >= 0))
    cell = np.broadcast_to(yi[:, :, None] * BEV + xi[:, None, :],
                           (D, H, W)).reshape(-1)
    vmask = valid.reshape(-1)
    pts = np.nonzero(vmask)[0]
    order = np.argsort(cell[pts], kind="stable")
    perm = pts[order].astype(np.int32)
    scell = cell[pts][order].astype(np.int32)
    nv = perm.size

    split_cell = int(scell[nv // 2])
    i0 = int(np.searchsorted(scell, split_cell, side="left"))
    nr0, nr1 = split_cell, NCELL - split_cell
    nra = max(nr0, nr1) + 1
    dummy = nra - 1

    lists = []
    for base, cnt, cell_base in ((0, i0, 0), (i0, nv - i0, split_cell)):
        q, r = divmod(cnt, NSUB)
        off = base
        for s in range(NSUB):
            take = q + (1 if s < r else 0)
            lists.append((perm[off:off + take],
                          scell[off:off + take] - cell_base))
            off += take
    nchunk = max(1, math.ceil(max(len(g) for g, _ in lists) / K))
    nchunk = math.ceil(nchunk / NBUF) * NBUF
    gidx = np.zeros((2 * NSUB, nchunk * K), np.int32)
    sidx = np.full((2 * NSUB, nchunk * K), dummy, np.int32)
    for w_, (g, sc_) in enumerate(lists):
        gidx[w_, :g.size] = g
        sidx[w_, :sc_.size] = sc_
    gidx = gidx.reshape(2 * NSUB * nchunk, K)
    sidx = sidx.reshape(2 * NSUB * nchunk, K)
    return gidx, sidx, nchunk, nr0, nr1, nra, split_cell


_GIDX, _SIDX, NCHUNK, NR0, NR1, NRA, SPLIT = _build_consts()
RPZ = math.ceil(NRA / NSUB)
RPS0 = math.ceil(NR0 / NSUB)
RPS1 = math.ceil(NR1 / NSUB)


def _tc_body(dl_ref, im_ref, f_ref):
    j = pl.program_id(1)
    lo = dl_ref[...]
    m = jnp.max(lo, axis=1, keepdims=True)
    e = jnp.exp(lo - m)
    r = 1.0 / jnp.sum(e, axis=1, keepdims=True)
    lo_j = dl_ref[:, pl.ds(j * DB, DB), :]
    p = jnp.exp(lo_j - m) * r
    im = im_ref[...]
    acc = p[0][:, :, None] * im[0][None, :, :]
    for n in range(1, N):
        acc = acc + p[n][:, :, None] * im[n][None, :, :]
    f_ref[...] = acc.reshape(DB, HWB * C)


def _tc_features(dl3, imt3):
    return pl.pallas_call(
        _tc_body,
        grid=(HW // HWB, D // DB),
        in_specs=[
            pl.BlockSpec((N, D, HWB), lambda i, j: (0, 0, i)),
            pl.BlockSpec((N, HWB, C), lambda i, j: (0, i, 0)),
        ],
        out_specs=pl.BlockSpec((DB, HWB * C), lambda i, j: (j, i)),
        out_shape=jax.ShapeDtypeStruct((D, HW * C), jnp.float32),
    )(dl3, imt3)


@functools.cache
def _make_sc_scatter():
    @functools.partial(
        pl.kernel,
        mesh=plsc.VectorSubcoreMesh(core_axis_name="c", subcore_axis_name="s"),
        out_type=jax.ShapeDtypeStruct((NCELL, C), jnp.float32),
        scratch_types=[
            pltpu.VMEM((NCHUNK, K), jnp.int32),
            pltpu.VMEM((NCHUNK, K), jnp.int32),
            pltpu.VMEM((NBUF, K, C), jnp.float32),
            pltpu.VMEM_SHARED((NRA, C), jnp.float32),
            pltpu.SemaphoreType.DMA,
            pltpu.SemaphoreType.DMA,
            pltpu.SemaphoreType.DMA,
            pltpu.SemaphoreType.DMA,
            pltpu.SemaphoreType.DMA,
            pltpu.SemaphoreType.DMA,
            pltpu.SemaphoreType.DMA,
            pltpu.SemaphoreType.DMA,
        ],
        compiler_params=pltpu.CompilerParams(use_tc_tiling_on_sc=False),
    )
    def _sc_scatter(f_hbm, gidx_hbm, sidx_hbm, zrows_hbm, out_hbm,
                    gidx_all, sidx_all, rows, acc,
                    g0, g1, g2, g3, t0, t1, t2, t3):
        sems = [g0, g1, g2, g3]
        ssems = [t0, t1, t2, t3]
        c = lax.axis_index("c")
        s = lax.axis_index("s")
        wid = c * NSUB + s

        pltpu.sync_copy(gidx_hbm.at[pl.ds(wid * NCHUNK, NCHUNK)], gidx_all)
        pltpu.sync_copy(sidx_hbm.at[pl.ds(wid * NCHUNK, NCHUNK)], sidx_all)

        def gather(j, b):
            return pltpu.make_async_copy(f_hbm.at[gidx_all.at[j]],
                                         rows.at[b], sems[b])

        for b in range(NBUF):
            gather(b, b).start()

        zst = jnp.minimum(s * RPZ, NRA - RPZ)
        pltpu.sync_copy(zrows_hbm, acc.at[pl.ds(zst, RPZ)])
        plsc.subcore_barrier()

        def scatter_wait(j, b):
            pltpu.make_async_copy(rows.at[b], acc.at[sidx_all.at[j]],
                                  ssems[b]).wait()

        @pl.loop(0, NCHUNK, step=NBUF)
        def _(j0):
            for b in range(NBUF):
                j = j0 + b
                gather(j, b).wait()
                pltpu.async_copy(rows.at[b], acc.at[sidx_all.at[j]],
                                 ssems[b], add=True)
                pj = j - 1
                pb = (b - 1) % NBUF

                @pl.when(pj >= 0)
                def _():
                    scatter_wait(pj, pb)

                @pl.when(jnp.logical_and(pj >= 0, pj + NBUF < NCHUNK))
                def _():
                    gather(pj + NBUF, pb).start()

        scatter_wait(NCHUNK - 1, (NCHUNK - 1) % NBUF)
        plsc.subcore_barrier()

        @pl.when(c == 0)
        def _():
            st = jnp.minimum(s * RPS0, NR0 - RPS0)
            pltpu.sync_copy(acc.at[pl.ds(st, RPS0)],
                            out_hbm.at[pl.ds(st, RPS0)])

        @pl.when(c == 1)
        def _():
            st = jnp.minimum(s * RPS1, NR1 - RPS1)
            pltpu.sync_copy(acc.at[pl.ds(st, RPS1)],
                            out_hbm.at[pl.ds(SPLIT + st, RPS1)])

    return _sc_scatter


def kernel(img_feat, depth_logits, rots, trans, intrins):
    del rots, trans, intrins
    imf = img_feat.reshape(N, C, H, W)
    dl3 = depth_logits.reshape(N, D, HW)
    imt3 = jnp.transpose(imf, (0, 2, 3, 1)).reshape(N, HW, C)
    feats = jnp.reshape(_tc_features(dl3, imt3), (NP, C))
    bev = _make_sc_scatter()(feats, jnp.asarray(_GIDX), jnp.asarray(_SIDX),
                             jnp.zeros((RPZ, C), jnp.float32))
    return jnp.transpose(bev, (1, 0)).reshape(1, C, BEV, BEV)


# speedup vs baseline: 9.2954x; 1.0234x over previous
import functools
import math

import numpy as np
import jax
import jax.numpy as jnp
from jax import lax
from jax.experimental import pallas as pl
from jax.experimental.pallas import tpu as pltpu
from jax.experimental.pallas import tpu_sc as plsc

N, C, D, H, W = 6, 64, 40, 32, 88
HW = H * W
NP = D * HW
BEV = 125
NCELL = BEV * BEV
K = 128
NBUF = 8
LOOKAHEAD = 4
NSUB = 16
HWB = 256
DB = 8


def _build_consts():
    z = np.arange(1.0, 41.0, 1.0, dtype=np.float32) + np.float32(1.0)
    x = np.linspace(0.0, W - 1, W).astype(np.float32)
    y = np.linspace(0.0, H - 1, H).astype(np.float32)
    xz = x[None, :] * z[:, None]
    yz = y[None, :] * z[:, None]
    m1 = np.float32(max(xz.max(), yz.max(), z.max()))
    col0 = xz / m1 * np.float32(100.0) + np.float32(-50.0)
    m2 = np.float32(max(col0.max(), yz.max(), z.max()))
    col1 = yz / m2 * np.float32(100.0) + np.float32(-50.0)
    xi = ((col0 - np.float32(-50.0)) / np.float32(0.8)).astype(np.int32)
    yi = ((col1 - np.float32(-50.0)) / np.float32(0.8)).astype(np.int32)
    zi = ((z - np.float32(-10.0)) / np.float32(20.0)).astype(np.int32)
    valid = ((xi[:, None, :] >= 0) & (xi[:, None, :] < BEV)
             & (yi[:, :, None] >= 0) & (yi[:, :, None] < BEV)
             & (zi[:, None, None] >= 0))
    cell = np.broadcast_to(yi[:, :, None] * BEV + xi[:, None, :],
                           (D, H, W)).reshape(-1)
    vmask = valid.reshape(-1)
    pts = np.nonzero(vmask)[0]
    order = np.argsort(cell[pts], kind="stable")
    perm = pts[order].astype(np.int32)
    scell = cell[pts][order].astype(np.int32)
    nv = perm.size

    split_cell = int(scell[nv // 2])
    i0 = int(np.searchsorted(scell, split_cell, side="left"))
    nr0, nr1 = split_cell, NCELL - split_cell
    nra = max(nr0, nr1) + 1
    dummy = nra - 1

    lists = []
    for base, cnt, cell_base in ((0, i0, 0), (i0, nv - i0, split_cell)):
        q, r = divmod(cnt, NSUB)
        off = base
        for s in range(NSUB):
            take = q + (1 if s < r else 0)
            lists.append((perm[off:off + take],
                          scell[off:off + take] - cell_base))
            off += take
    nchunk = max(NBUF, math.ceil(max(len(g) for g, _ in lists) / K))
    nchunk = math.ceil(nchunk / LOOKAHEAD) * LOOKAHEAD
    gidx = np.zeros((2 * NSUB, nchunk * K), np.int32)
    sidx = np.full((2 * NSUB, nchunk * K), dummy, np.int32)
    for w_, (g, sc_) in enumerate(lists):
        gidx[w_, :g.size] = g
        sidx[w_, :sc_.size] = sc_
    gidx = gidx.reshape(2 * NSUB * nchunk, K)
    sidx = sidx.reshape(2 * NSUB * nchunk, K)
    return gidx, sidx, nchunk, nr0, nr1, nra, split_cell


_GIDX, _SIDX, NCHUNK, NR0, NR1, NRA, SPLIT = _build_consts()
RPZ = math.ceil(NRA / NSUB)
RPS0 = math.ceil(NR0 / NSUB)
RPS1 = math.ceil(NR1 / NSUB)


SMB = 1408


def _sm_body(dl_ref, dp_ref):
    lo = dl_ref[...]
    m = jnp.max(lo, axis=1, keepdims=True)
    e = jnp.exp(lo - m)
    dp_ref[...] = e / jnp.sum(e, axis=1, keepdims=True)


def _tc_body(dp_ref, im_ref, f_ref):
    p = dp_ref[...]
    im = im_ref[...]
    acc = p[0][:, :, None] * im[0][None, :, :]
    for n in range(1, N):
        acc = acc + p[n][:, :, None] * im[n][None, :, :]
    f_ref[...] = acc.reshape(DB, HWB * C)


def _tc_features(dl3, imt3):
    dp = pl.pallas_call(
        _sm_body,
        grid=(HW // SMB,),
        in_specs=[pl.BlockSpec((N, D, SMB), lambda i: (0, 0, i))],
        out_specs=pl.BlockSpec((N, D, SMB), lambda i: (0, 0, i)),
        out_shape=jax.ShapeDtypeStruct((N, D, HW), jnp.float32),
    )(dl3)
    return pl.pallas_call(
        _tc_body,
        grid=(HW // HWB, D // DB),
        in_specs=[
            pl.BlockSpec((N, DB, HWB), lambda i, j: (0, j, i)),
            pl.BlockSpec((N, HWB, C), lambda i, j: (0, i, 0)),
        ],
        out_specs=pl.BlockSpec((DB, HWB * C), lambda i, j: (j, i)),
        out_shape=jax.ShapeDtypeStruct((D, HW * C), jnp.float32),
    )(dp, imt3)


@functools.cache
def _make_sc_scatter():
    @functools.partial(
        pl.kernel,
        mesh=plsc.VectorSubcoreMesh(core_axis_name="c", subcore_axis_name="s"),
        out_type=jax.ShapeDtypeStruct((NCELL, C), jnp.float32),
        scratch_types=[
            pltpu.VMEM((NCHUNK, K), jnp.int32),
            pltpu.VMEM((NCHUNK, K), jnp.int32),
            pltpu.VMEM((NBUF, K, C), jnp.float32),
            pltpu.VMEM_SHARED((NRA, C), jnp.float32),
        ] + [pltpu.SemaphoreType.DMA] * (2 * NBUF),
        compiler_params=pltpu.CompilerParams(use_tc_tiling_on_sc=False),
    )
    def _sc_scatter(f_hbm, gidx_hbm, sidx_hbm, zrows_hbm, out_hbm,
                    gidx_all, sidx_all, rows, acc,
                    g0, g1, g2, g3, g4, g5, g6, g7,
                    t0, t1, t2, t3, t4, t5, t6, t7):
        sems = [g0, g1, g2, g3, g4, g5, g6, g7]
        ssems = [t0, t1, t2, t3, t4, t5, t6, t7]
        c = lax.axis_index("c")
        s = lax.axis_index("s")
        wid = c * NSUB + s

        pltpu.sync_copy(gidx_hbm.at[pl.ds(wid * NCHUNK, NCHUNK)], gidx_all)
        pltpu.sync_copy(sidx_hbm.at[pl.ds(wid * NCHUNK, NCHUNK)], sidx_all)

        def gather(j, b):
            return pltpu.make_async_copy(f_hbm.at[gidx_all.at[j]],
                                         rows.at[b], sems[b])

        for b in range(NBUF):
            gather(b, b).start()

        zst = jnp.minimum(s * RPZ, NRA - RPZ)
        pltpu.sync_copy(zrows_hbm, acc.at[pl.ds(zst, RPZ)])
        plsc.subcore_barrier()

        def scatter_wait(j, b):
            pltpu.make_async_copy(rows.at[b], acc.at[sidx_all.at[j]],
                                  ssems[b]).wait()

        def stage(j, b):
            gather(j, b).wait()
            pltpu.async_copy(rows.at[b], acc.at[sidx_all.at[j]],
                             ssems[b], add=True)
            pj = j - LOOKAHEAD
            pb = (b + LOOKAHEAD) % NBUF

            @pl.when(pj >= jnp.int32(0))
            def _():
                scatter_wait(pj, pb)

            @pl.when(jnp.logical_and(pj >= jnp.int32(0),
                                     pj + NBUF < NCHUNK))
            def _():
                gather(pj + NBUF, pb).start()

        bulk = (NCHUNK // NBUF) * NBUF

        @pl.loop(0, bulk, step=NBUF)
        def _(j0):
            for b in range(NBUF):
                stage(j0 + b, b)

        for j in range(bulk, NCHUNK):
            stage(j, j % NBUF)
        for j in range(NCHUNK - LOOKAHEAD, NCHUNK):
            scatter_wait(j, j % NBUF)
        plsc.subcore_barrier()

        @pl.when(c == 0)
        def _():
            st = jnp.minimum(s * RPS0, NR0 - RPS0)
            pltpu.sync_copy(acc.at[pl.ds(st, RPS0)],
                            out_hbm.at[pl.ds(st, RPS0)])

        @pl.when(c == 1)
        def _():
            st = jnp.minimum(s * RPS1, NR1 - RPS1)
            pltpu.sync_copy(acc.at[pl.ds(st, RPS1)],
                            out_hbm.at[pl.ds(SPLIT + st, RPS1)])

    return _sc_scatter


def kernel(img_feat, depth_logits, rots, trans, intrins):
    del rots, trans, intrins
    imf = img_feat.reshape(N, C, H, W)
    dl3 = depth_logits.reshape(N, D, HW)
    imt3 = jnp.transpose(imf, (0, 2, 3, 1)).reshape(N, HW, C)
    feats = jnp.reshape(_tc_features(dl3, imt3), (NP, C))
    bev = _make_sc_scatter()(feats, jnp.asarray(_GIDX), jnp.asarray(_SIDX),
                             jnp.zeros((RPZ, C), jnp.float32))
    return jnp.transpose(bev, (1, 0)).reshape(1, C, BEV, BEV)
